# Initial kernel scaffold; baseline (speedup 1.0000x reference)
import jax
import jax.numpy as jnp
from jax.experimental import pallas as pl


def kernel(z, e):
    raise NotImplementedError("write your pallas kernel here")


# speedup vs baseline: 3.1195x; 3.1195x over previous
import functools

import jax
import jax.numpy as jnp
from jax import lax
from jax.experimental import pallas as pl
from jax.experimental.pallas import tpu as pltpu
from jax.experimental.pallas import tpu_sc as plsc

_B = 320000
_D = 128
_L = 16
_NC = 2
_NS = 16
_NW = _NC * _NS
_PER_W = _B // _NW
_C = 400
_NCHUNK = _PER_W // _C


def _tec_body(z_hbm, e0_hbm, e1_hbm, out_hbm,
              idx0_v, idx1_v, rows0_v, rows1_v, res_v, sem0, sem1):
    wid = lax.axis_index("s") * _NC + lax.axis_index("c")
    base = wid * _PER_W

    def chunk(i, _):
        cb = base + i * _C
        pltpu.sync_copy(e0_hbm.at[pl.ds(cb, _C)], idx0_v)
        pltpu.sync_copy(e1_hbm.at[pl.ds(cb, _C)], idx1_v)
        g0 = pltpu.async_copy(z_hbm.at[idx0_v], rows0_v, sem0)
        g1 = pltpu.async_copy(z_hbm.at[idx1_v], rows1_v, sem1)
        g0.wait()
        g1.wait()

        lanes = lax.iota(jnp.int32, _L)

        def group(g, _):
            base_c = g * _L
            tot = jnp.zeros((_L,), jnp.float32)
            for e2 in range(_L):
                c = base_c + e2
                acc = rows0_v[c, pl.ds(0, _L)] * rows1_v[c, pl.ds(0, _L)]
                for l in range(1, _D // _L):
                    acc = acc + (rows0_v[c, pl.ds(l * _L, _L)]
                                 * rows1_v[c, pl.ds(l * _L, _L)])
                s = jnp.sum(acc)
                tot = jnp.where(lanes == e2, s, tot)
            res_v[pl.ds(base_c, _L)] = 1.0 / (1.0 + jnp.exp(-tot))
            return ()

        lax.fori_loop(0, _C // _L, group, ())
        pltpu.sync_copy(res_v, out_hbm.at[pl.ds(cb, _C)])
        return ()

    lax.fori_loop(0, _NCHUNK, chunk, ())


@functools.partial(jax.jit, static_argnums=())
def _sc_call(z, e0, e1):
    mesh = plsc.VectorSubcoreMesh(core_axis_name="c", subcore_axis_name="s")
    f = pl.kernel(
        _tec_body,
        mesh=mesh,
        compiler_params=pltpu.CompilerParams(needs_layout_passes=False),
        out_type=jax.ShapeDtypeStruct((_B,), jnp.float32),
        scratch_types=[
            pltpu.VMEM((_C,), jnp.int32),
            pltpu.VMEM((_C,), jnp.int32),
            pltpu.VMEM((_C, _D), jnp.float32),
            pltpu.VMEM((_C, _D), jnp.float32),
            pltpu.VMEM((_C,), jnp.float32),
            pltpu.SemaphoreType.DMA,
            pltpu.SemaphoreType.DMA,
        ],
    )
    return f(z, e0, e1)


def kernel(z, e):
    e0 = e[0].astype(jnp.int32)
    e1 = e[1].astype(jnp.int32)
    return _sc_call(z, e0, e1)


# speedup vs baseline: 4.0865x; 1.3100x over previous
import functools

import jax
import jax.numpy as jnp
from jax import lax
from jax.experimental import pallas as pl
from jax.experimental.pallas import tpu as pltpu
from jax.experimental.pallas import tpu_sc as plsc

_B = 320000
_D = 128
_L = 16
_NC = 2
_NS = 16
_NW = _NC * _NS
_PER_W = _B // _NW
_C = 80
_NCHUNK = _PER_W // _C


def _tec_body(z_hbm, e0_hbm, e1_hbm, out_hbm,
              idx0_f, idx1_f, rows0_a, rows1_a, rows0_b, rows1_b, res_f,
              sa0, sa1, sb0, sb1):
    wid = lax.axis_index("s") * _NC + lax.axis_index("c")
    base = wid * _PER_W

    pltpu.sync_copy(e0_hbm.at[pl.ds(base, _PER_W)], idx0_f)
    pltpu.sync_copy(e1_hbm.at[pl.ds(base, _PER_W)], idx1_f)

    lanes = lax.iota(jnp.int32, _L)

    def issue(i, r0, r1, s0, s1):
        off = i * _C
        pltpu.async_copy(z_hbm.at[idx0_f.at[pl.ds(off, _C)]], r0, s0)
        pltpu.async_copy(z_hbm.at[idx1_f.at[pl.ds(off, _C)]], r1, s1)

    def wait(i, r0, r1, s0, s1):
        off = i * _C
        pltpu.make_async_copy(z_hbm.at[idx0_f.at[pl.ds(off, _C)]], r0, s0).wait()
        pltpu.make_async_copy(z_hbm.at[idx1_f.at[pl.ds(off, _C)]], r1, s1).wait()

    def compute(i, r0, r1):
        def group(g, _):
            base_c = g * _L
            tot = jnp.zeros((_L,), jnp.float32)
            for e2 in range(_L):
                c = base_c + e2
                acc = r0[c, pl.ds(0, _L)] * r1[c, pl.ds(0, _L)]
                for l in range(1, _D // _L):
                    acc = acc + (r0[c, pl.ds(l * _L, _L)]
                                 * r1[c, pl.ds(l * _L, _L)])
                s = jnp.sum(acc)
                tot = jnp.where(lanes == e2, s, tot)
            res_f[pl.ds(i * _C + base_c, _L)] = 1.0 / (1.0 + jnp.exp(-tot))
            return ()

        lax.fori_loop(0, _C // _L, group, ())

    issue(0, rows0_a, rows1_a, sa0, sa1)
    issue(1, rows0_b, rows1_b, sb0, sb1)

    def body(j, _):
        i0 = 2 * j
        i1 = 2 * j + 1
        wait(i0, rows0_a, rows1_a, sa0, sa1)
        compute(i0, rows0_a, rows1_a)
        issue(i0 + 2, rows0_a, rows1_a, sa0, sa1)
        wait(i1, rows0_b, rows1_b, sb0, sb1)
        compute(i1, rows0_b, rows1_b)

        @pl.when(i1 + 2 < _NCHUNK)
        def _():
            issue(i1 + 2, rows0_b, rows1_b, sb0, sb1)

        return ()

    lax.fori_loop(0, _NCHUNK // 2, body, ())

    wait(_NCHUNK - 1, rows0_a, rows1_a, sa0, sa1)
    compute(_NCHUNK - 1, rows0_a, rows1_a)
    pltpu.sync_copy(res_f, out_hbm.at[pl.ds(base, _PER_W)])


@functools.partial(jax.jit, static_argnums=())
def _sc_call(z, e0, e1):
    mesh = plsc.VectorSubcoreMesh(core_axis_name="c", subcore_axis_name="s")
    f = pl.kernel(
        _tec_body,
        mesh=mesh,
        compiler_params=pltpu.CompilerParams(needs_layout_passes=False),
        out_type=jax.ShapeDtypeStruct((_B,), jnp.float32),
        scratch_types=[
            pltpu.VMEM((_PER_W,), jnp.int32),
            pltpu.VMEM((_PER_W,), jnp.int32),
            pltpu.VMEM((_C, _D), jnp.float32),
            pltpu.VMEM((_C, _D), jnp.float32),
            pltpu.VMEM((_C, _D), jnp.float32),
            pltpu.VMEM((_C, _D), jnp.float32),
            pltpu.VMEM((_PER_W,), jnp.float32),
            pltpu.SemaphoreType.DMA,
            pltpu.SemaphoreType.DMA,
            pltpu.SemaphoreType.DMA,
            pltpu.SemaphoreType.DMA,
        ],
    )
    return f(z, e0, e1)


def kernel(z, e):
    e0 = e[0].astype(jnp.int32)
    e1 = e[1].astype(jnp.int32)
    return _sc_call(z, e0, e1)
